# Initial kernel scaffold; baseline (speedup 1.0000x reference)
import jax
import jax.numpy as jnp
from jax.experimental import pallas as pl


def kernel(src_v, src_f, tgt_points, noise):
    raise NotImplementedError("write your pallas kernel here")


# speedup vs baseline: 5.3685x; 5.3685x over previous
import functools

import jax
import jax.numpy as jnp
from jax import lax
from jax.experimental import pallas as pl

_V = 1024
_F = 2048
_T = 4096
_UP = 3
_K = 5
_STD = 0.05
_Q = _T * _UP + _F
_BQ = 128

_f32 = jnp.float32
_HI = lax.Precision.HIGHEST


def _prep_kernel(src_v_ref, src_ft_ref, ftab_ref, centert_ref):
    src_v = src_v_ref[:, :]
    fts = []
    for j in range(3):
        fid = src_ft_ref[j : j + 1, :]
        vio = lax.broadcasted_iota(jnp.int32, (_V, _F), 0)
        oh = (vio == fid).astype(_f32)
        fjT = lax.dot_general(
            src_v,
            oh,
            (((0,), (0,)), ((), ())),
            precision=_HI,
            preferred_element_type=_f32,
        )
        fts.append(fjT)
        ftab_ref[3 * j : 3 * j + 3, :] = fjT
    ftab_ref[9:16, :] = jnp.zeros((7, _F), _f32)
    centert = (fts[0] + fts[1] + fts[2]) / 3.0
    centert_ref[0:3, :] = centert
    centert_ref[3:8, :] = jnp.zeros((5, _F), _f32)


def _main_kernel(qp_ref, ftab_ref, tgtt_ref, tgt_ref, out_ref):
    i = pl.program_id(0)

    p = qp_ref[:, :]
    px = p[:, 0:1]
    py = p[:, 1:2]
    pz = p[:, 2:3]

    ax = ftab_ref[0:1, :]
    ay = ftab_ref[1:2, :]
    az = ftab_ref[2:3, :]
    bx = ftab_ref[3:4, :]
    by = ftab_ref[4:5, :]
    bz = ftab_ref[5:6, :]
    cx = ftab_ref[6:7, :]
    cy = ftab_ref[7:8, :]
    cz = ftab_ref[8:9, :]

    abx = bx - ax
    aby = by - ay
    abz = bz - az
    acx = cx - ax
    acy = cy - ay
    acz = cz - az

    apx = px - ax
    apy = py - ay
    apz = pz - az
    d1 = (abx * apx + aby * apy) + abz * apz
    d2_ = (acx * apx + acy * apy) + acz * apz

    bpx = px - bx
    bpy = py - by
    bpz = pz - bz
    d3 = (abx * bpx + aby * bpy) + abz * bpz
    d4 = (acx * bpx + acy * bpy) + acz * bpz

    cpx = px - cx
    cpy = py - cy
    cpz = pz - cz
    d5 = (abx * cpx + aby * cpy) + abz * cpz
    d6 = (acx * cpx + acy * cpy) + acz * cpz

    vc = d1 * d4 - d3 * d2_
    vb = d5 * d2_ - d1 * d6
    va = d3 * d6 - d5 * d4
    eps = 1e-12
    denom = va + vb + vc
    v = vb / (denom + eps)
    w = vc / (denom + eps)
    u = 1.0 - v - w
    tbc = (d4 - d3) / ((d4 - d3) + (d5 - d6) + eps)
    m = (va <= 0) & ((d4 - d3) >= 0) & ((d5 - d6) >= 0)
    u = jnp.where(m, 0.0, u)
    v = jnp.where(m, 1.0 - tbc, v)
    w = jnp.where(m, tbc, w)
    tac = d2_ / (d2_ - d6 + eps)
    m = (vb <= 0) & (d2_ >= 0) & (d6 <= 0)
    u = jnp.where(m, 1.0 - tac, u)
    v = jnp.where(m, 0.0, v)
    w = jnp.where(m, tac, w)
    m = (d6 >= 0) & (d5 <= d6)
    u = jnp.where(m, 0.0, u)
    v = jnp.where(m, 0.0, v)
    w = jnp.where(m, 1.0, w)
    tab = d1 / (d1 - d3 + eps)
    m = (vc <= 0) & (d1 >= 0) & (d3 <= 0)
    u = jnp.where(m, 1.0 - tab, u)
    v = jnp.where(m, tab, v)
    w = jnp.where(m, 0.0, w)
    m = (d3 >= 0) & (d4 <= d3)
    u = jnp.where(m, 0.0, u)
    v = jnp.where(m, 1.0, v)
    w = jnp.where(m, 0.0, w)
    m = (d1 <= 0) & (d2_ <= 0)
    u = jnp.where(m, 1.0, u)
    v = jnp.where(m, 0.0, v)
    w = jnp.where(m, 0.0, w)

    clx = (u * ax + v * bx) + w * cx
    cly = (u * ay + v * by) + w * cy
    clz = (u * az + v * bz) + w * cz
    ddx = px - clx
    ddy = py - cly
    ddz = pz - clz
    dsq = (ddx * ddx + ddy * ddy) + ddz * ddz

    ids_f = lax.broadcasted_iota(jnp.int32, (_BQ, _F), 1)
    mn = jnp.min(dsq, axis=1, keepdims=True)
    fi = jnp.min(jnp.where(dsq == mn, ids_f, _F), axis=1, keepdims=True)
    msk = ids_f == fi
    bcx = jnp.sum(jnp.where(msk, clx, 0.0), axis=1, keepdims=True)
    bcy = jnp.sum(jnp.where(msk, cly, 0.0), axis=1, keepdims=True)
    bcz = jnp.sum(jnp.where(msk, clz, 0.0), axis=1, keepdims=True)

    dsx = px - bcx
    dsy = py - bcy
    dsz = pz - bcz
    t0 = dsx + 1e-10
    t1 = dsy + 1e-10
    t2 = dsz + 1e-10
    udf_s = jnp.sqrt((t0 * t0 + t1 * t1) + t2 * t2)

    tx = tgtt_ref[0:1, :]
    ty = tgtt_ref[1:2, :]
    tz = tgtt_ref[2:3, :]
    dxt = px - tx
    dyt = py - ty
    dzt = pz - tz
    d2t = (dxt * dxt + dyt * dyt) + dzt * dzt

    ids_t = lax.broadcasted_iota(jnp.int32, (_BQ, _T), 1)
    run = d2t
    mw = jnp.zeros((_BQ, _T), _f32)
    s = jnp.zeros((_BQ, 1), _f32)
    for k in range(_K):
        mk = jnp.min(run, axis=1, keepdims=True)
        fik = jnp.min(jnp.where(run == mk, ids_t, _T), axis=1, keepdims=True)
        mskk = ids_t == fik
        wk = 1.0 / (mk + 1e-8)
        mw = mw + jnp.where(mskk, wk, 0.0)
        s = s + wk
        if k < _K - 1:
            run = jnp.where(mskk, jnp.inf, run)

    spts = lax.dot_general(
        mw,
        tgt_ref[:, :],
        (((1,), (0,)), ((), ())),
        precision=_HI,
        preferred_element_type=_f32,
    )
    dtx = px - spts[:, 0:1] / s
    dty = py - spts[:, 1:2] / s
    dtz = pz - spts[:, 2:3] / s
    r0 = dtx + 1e-10
    r1 = dty + 1e-10
    r2 = dtz + 1e-10
    udf_t = jnp.sqrt((r0 * r0 + r1 * r1) + r2 * r2)

    err = (
        (jnp.abs(dsx - dtx) + jnp.abs(dsy - dty)) + jnp.abs(dsz - dtz)
    ) + jnp.abs(udf_s - udf_t)
    tot = jnp.sum(err)

    ri = lax.broadcasted_iota(jnp.int32, (8, 128), 0)
    ci = lax.broadcasted_iota(jnp.int32, (8, 128), 1)
    upd = jnp.where((ri == 0) & (ci == 0), tot, 0.0)

    @pl.when(i == 0)
    def _():
        out_ref[:, :] = jnp.zeros((8, 128), _f32)

    out_ref[:, :] += upd


@functools.partial(jax.jit, static_argnames=("interpret",))
def kernel(src_v, src_f, tgt_points, noise, interpret=False):
    src_ft = jnp.zeros((8, _F), jnp.int32).at[0:3, :].set(src_f.T)

    ftab, centert = pl.pallas_call(
        _prep_kernel,
        out_shape=[
            jax.ShapeDtypeStruct((16, _F), _f32),
            jax.ShapeDtypeStruct((8, _F), _f32),
        ],
        interpret=interpret,
    )(src_v, src_ft)

    center = centert[0:3, :].T
    qp = jnp.concatenate(
        [(tgt_points[:, None, :] + _STD * noise).reshape(-1, 3), center], axis=0
    )
    tgtt = jnp.zeros((8, _T), _f32).at[0:3, :].set(tgt_points.T)

    acc = pl.pallas_call(
        _main_kernel,
        grid=(_Q // _BQ,),
        in_specs=[
            pl.BlockSpec((_BQ, 3), lambda i: (i, 0)),
            pl.BlockSpec((16, _F), lambda i: (0, 0)),
            pl.BlockSpec((8, _T), lambda i: (0, 0)),
            pl.BlockSpec((_T, 3), lambda i: (0, 0)),
        ],
        out_specs=pl.BlockSpec((8, 128), lambda i: (0, 0)),
        out_shape=jax.ShapeDtypeStruct((8, 128), _f32),
        interpret=interpret,
    )(qp, ftab, tgtt, tgt_points)

    return acc[0, 0] / _Q


# speedup vs baseline: 6.9858x; 1.3013x over previous
import functools

import jax
import jax.numpy as jnp
from jax import lax
from jax.experimental import pallas as pl

_V = 1024
_F = 2048
_T = 4096
_UP = 3
_K = 5
_STD = 0.05
_Q = _T * _UP + _F
_BQ = 128
_G = 1024
_NO = _T // _G

_f32 = jnp.float32
_i32 = jnp.int32
_HI = lax.Precision.HIGHEST
_IMAX = jnp.iinfo(jnp.int32).max


def _prep_kernel(src_v_ref, src_ft_ref, ftab_ref, gmat_ref):
    src_v = src_v_ref[:, :]
    fts = []
    for j in range(3):
        fid = src_ft_ref[j : j + 1, :]
        vio = lax.broadcasted_iota(_i32, (_V, _F), 0)
        oh = (vio == fid).astype(_f32)
        fjT = lax.dot_general(
            src_v,
            oh,
            (((0,), (0,)), ((), ())),
            precision=_HI,
            preferred_element_type=_f32,
        )
        fts.append(fjT)
    a, b, c = fts
    ab = b - a
    ac = c - a
    ftab_ref[0:3, :] = a
    ftab_ref[3:6, :] = ab
    ftab_ref[6:9, :] = ac
    aa = jnp.sum(ab * ab, axis=0, keepdims=True)
    e = jnp.sum(ab * ac, axis=0, keepdims=True)
    cc = jnp.sum(ac * ac, axis=0, keepdims=True)
    gce = cc - e
    gae = aa - e
    ftab_ref[9:10, :] = jnp.sum(ab * a, axis=0, keepdims=True)
    ftab_ref[10:11, :] = jnp.sum(ac * a, axis=0, keepdims=True)
    ftab_ref[11:12, :] = aa
    ftab_ref[12:13, :] = e
    ftab_ref[13:14, :] = cc
    ftab_ref[14:15, :] = gce
    ftab_ref[15:16, :] = gae
    ftab_ref[16:17, :] = gae * cc + e * gce
    ftab_ref[17:24, :] = jnp.zeros((7, _F), _f32)
    gmat_ref[0:3, 0:_F] = ab
    gmat_ref[0:3, _F : 2 * _F] = ac
    gmat_ref[3:8, :] = jnp.zeros((5, 2 * _F), _f32)


def _main_kernel(qp_ref, ftab_ref, gmat_ref, tgtt_ref, tgq_ref, out_ref):
    i = pl.program_id(0)

    p = qp_ref[:, :]
    px = p[:, 0:1]
    py = p[:, 1:2]
    pz = p[:, 2:3]

    gres = lax.dot_general(
        p,
        gmat_ref[0:3, :],
        (((1,), (0,)), ((), ())),
        precision=_HI,
        preferred_element_type=_f32,
    )
    d1 = gres[:, 0:_F] - ftab_ref[9:10, :]
    d2_ = gres[:, _F : 2 * _F] - ftab_ref[10:11, :]
    aa = ftab_ref[11:12, :]
    e = ftab_ref[12:13, :]
    cc = ftab_ref[13:14, :]
    gce = ftab_ref[14:15, :]
    gae = ftab_ref[15:16, :]
    det = ftab_ref[16:17, :]
    d3 = d1 - aa
    d4 = d2_ - e
    d5 = d1 - e
    d6 = d2_ - cc

    d21 = d2_ - d1
    vb = gce * d1 - e * d21
    vc = gae * d2_ + e * d21
    va = det - (gce * d1 + gae * d2_)
    eps = 1e-12
    rcp = 1.0 / ((va + vb) + vc + eps)
    v = vb * rcp
    w = vc * rcp
    s1 = d4 - d3
    s2 = d5 - d6
    tbc = s1 / ((s1 + s2) + eps)
    m = (va <= 0) & (s1 >= 0) & (s2 >= 0)
    v = jnp.where(m, 1.0 - tbc, v)
    w = jnp.where(m, tbc, w)
    tac = d2_ / ((d2_ - d6) + eps)
    m = (vb <= 0) & (d2_ >= 0) & (d6 <= 0)
    v = jnp.where(m, 0.0, v)
    w = jnp.where(m, tac, w)
    m = (d6 >= 0) & (d5 <= d6)
    v = jnp.where(m, 0.0, v)
    w = jnp.where(m, 1.0, w)
    tab = d1 / ((d1 - d3) + eps)
    m = (vc <= 0) & (d1 >= 0) & (d3 <= 0)
    v = jnp.where(m, tab, v)
    w = jnp.where(m, 0.0, w)
    m = (d3 >= 0) & (d4 <= d3)
    v = jnp.where(m, 1.0, v)
    w = jnp.where(m, 0.0, w)
    m = (d1 <= 0) & (d2_ <= 0)
    v = jnp.where(m, 0.0, v)
    w = jnp.where(m, 0.0, w)

    clx = ftab_ref[0:1, :] + (v * ftab_ref[3:4, :] + w * ftab_ref[6:7, :])
    cly = ftab_ref[1:2, :] + (v * ftab_ref[4:5, :] + w * ftab_ref[7:8, :])
    clz = ftab_ref[2:3, :] + (v * ftab_ref[5:6, :] + w * ftab_ref[8:9, :])
    ddx = px - clx
    ddy = py - cly
    ddz = pz - clz
    dsq = (ddx * ddx + ddy * ddy) + ddz * ddz

    ids_f = lax.broadcasted_iota(_i32, (_BQ, _F), 1)
    packed_a = jnp.bitwise_or(
        jnp.bitwise_and(lax.bitcast_convert_type(dsq, _i32), _i32(-2048)), ids_f
    )
    ma = jnp.min(packed_a, axis=1, keepdims=True)
    eqa = ids_f == jnp.bitwise_and(ma, _i32(2047))
    bcx = jnp.sum(jnp.where(eqa, clx, 0.0), axis=1, keepdims=True)
    bcy = jnp.sum(jnp.where(eqa, cly, 0.0), axis=1, keepdims=True)
    bcz = jnp.sum(jnp.where(eqa, clz, 0.0), axis=1, keepdims=True)

    dsx = px - bcx
    dsy = py - bcy
    dsz = pz - bcz
    t0 = dsx + 1e-10
    t1 = dsy + 1e-10
    t2 = dsz + 1e-10
    udf_s = jnp.sqrt((t0 * t0 + t1 * t1) + t2 * t2)

    gt = lax.dot_general(
        p,
        tgtt_ref[0:3, :],
        (((1,), (0,)), ((), ())),
        precision=_HI,
        preferred_element_type=_f32,
    )
    pn = (px * px + py * py) + pz * pz
    tx = tgtt_ref[0:1, :]
    ty = tgtt_ref[1:2, :]
    tz = tgtt_ref[2:3, :]
    tn = (tx * tx + ty * ty) + tz * tz
    d2t = (pn - 2.0 * gt) + tn

    ids_t = lax.broadcasted_iota(_i32, (_BQ, _T), 1)
    packed = jnp.bitwise_or(
        jnp.bitwise_and(lax.bitcast_convert_type(d2t, _i32), _i32(-4096)), ids_t
    )
    gm = jnp.minimum(
        jnp.minimum(packed[:, 0:_G], packed[:, _G : 2 * _G]),
        jnp.minimum(packed[:, 2 * _G : 3 * _G], packed[:, 3 * _G : 4 * _G]),
    )

    odiv = lax.broadcasted_iota(_i32, (_BQ, 3 * _NO), 1) // 3
    ids_g = lax.broadcasted_iota(_i32, (_BQ, _G), 1)
    acc = jnp.zeros((_BQ, 3 * _NO), _f32)
    s = jnp.zeros((_BQ, 1), _f32)
    for k in range(_K):
        mk = jnp.min(gm, axis=1, keepdims=True)
        eq = ids_g == jnp.bitwise_and(mk, _i32(_G - 1))
        g1 = eq.astype(_f32)
        tmpk = lax.dot_general(
            g1,
            tgq_ref[:, :],
            (((1,), (0,)), ((), ())),
            precision=_HI,
            preferred_element_type=_f32,
        )
        dk = lax.bitcast_convert_type(jnp.bitwise_and(mk, _i32(-4096)), _f32)
        wk = 1.0 / (dk + 1e-8)
        s = s + wk
        ok = jnp.right_shift(jnp.bitwise_and(mk, _i32(4095)), _i32(10))
        acc = acc + jnp.where(odiv == ok, tmpk * wk, 0.0)
        if k < _K - 1:
            gm = jnp.where(eq, _IMAX, gm)

    spx = ((acc[:, 0:1] + acc[:, 3:4]) + (acc[:, 6:7] + acc[:, 9:10])) / s
    spy = ((acc[:, 1:2] + acc[:, 4:5]) + (acc[:, 7:8] + acc[:, 10:11])) / s
    spz = ((acc[:, 2:3] + acc[:, 5:6]) + (acc[:, 8:9] + acc[:, 11:12])) / s
    dtx = px - spx
    dty = py - spy
    dtz = pz - spz
    r0 = dtx + 1e-10
    r1 = dty + 1e-10
    r2 = dtz + 1e-10
    udf_t = jnp.sqrt((r0 * r0 + r1 * r1) + r2 * r2)

    err = (
        (jnp.abs(dsx - dtx) + jnp.abs(dsy - dty)) + jnp.abs(dsz - dtz)
    ) + jnp.abs(udf_s - udf_t)
    tot = jnp.sum(err)

    ri = lax.broadcasted_iota(_i32, (8, 128), 0)
    ci = lax.broadcasted_iota(_i32, (8, 128), 1)
    upd = jnp.where((ri == 0) & (ci == 0), tot, 0.0)

    @pl.when(i == 0)
    def _():
        out_ref[:, :] = jnp.zeros((8, 128), _f32)

    out_ref[:, :] += upd


@functools.partial(jax.jit, static_argnames=("interpret",))
def kernel(src_v, src_f, tgt_points, noise, interpret=False):
    src_ft = jnp.zeros((8, _F), _i32).at[0:3, :].set(src_f.T)

    ftab, gmat = pl.pallas_call(
        _prep_kernel,
        out_shape=[
            jax.ShapeDtypeStruct((24, _F), _f32),
            jax.ShapeDtypeStruct((8, 2 * _F), _f32),
        ],
        interpret=interpret,
    )(src_v, src_ft)

    center = ftab[0:3, :] + (ftab[3:6, :] + ftab[6:9, :]) / 3.0
    qp = jnp.concatenate(
        [(tgt_points[:, None, :] + _STD * noise).reshape(-1, 3), center.T], axis=0
    )
    tgtt = jnp.zeros((8, _T), _f32).at[0:3, :].set(tgt_points.T)
    tgq = jnp.transpose(tgt_points.reshape(_NO, _G, 3), (1, 0, 2)).reshape(
        _G, 3 * _NO
    )

    acc = pl.pallas_call(
        _main_kernel,
        grid=(_Q // _BQ,),
        in_specs=[
            pl.BlockSpec((_BQ, 3), lambda i: (i, 0)),
            pl.BlockSpec((24, _F), lambda i: (0, 0)),
            pl.BlockSpec((8, 2 * _F), lambda i: (0, 0)),
            pl.BlockSpec((8, _T), lambda i: (0, 0)),
            pl.BlockSpec((_G, 3 * _NO), lambda i: (0, 0)),
        ],
        out_specs=pl.BlockSpec((8, 128), lambda i: (0, 0)),
        out_shape=jax.ShapeDtypeStruct((8, 128), _f32),
        interpret=interpret,
    )(qp, ftab, gmat, tgtt, tgq)

    return acc[0, 0] / _Q


# speedup vs baseline: 11.3385x; 1.6231x over previous
import functools

import jax
import jax.numpy as jnp
from jax import lax
from jax.experimental import pallas as pl

_V = 1024
_F = 2048
_T = 4096
_UP = 3
_K = 5
_STD = 0.05
_Q = _T * _UP + _F
_BQ = 128
_G = 1024
_NO = _T // _G

_f32 = jnp.float32
_i32 = jnp.int32
_HI = lax.Precision.HIGHEST
_IMAX = jnp.iinfo(jnp.int32).max


def _prep_kernel(src_v_ref, src_ft_ref, ftab_ref, gmat_ref):
    src_v = src_v_ref[:, :]
    fts = []
    for j in range(3):
        fid = src_ft_ref[j : j + 1, :]
        vio = lax.broadcasted_iota(_i32, (_V, _F), 0)
        oh = (vio == fid).astype(_f32)
        fjT = lax.dot_general(
            src_v,
            oh,
            (((0,), (0,)), ((), ())),
            precision=_HI,
            preferred_element_type=_f32,
        )
        fts.append(fjT)
    a, b, c = fts
    ab = b - a
    ac = c - a
    ftab_ref[0:3, :] = a
    ftab_ref[3:6, :] = ab
    ftab_ref[6:9, :] = ac
    aa = jnp.sum(ab * ab, axis=0, keepdims=True)
    e = jnp.sum(ab * ac, axis=0, keepdims=True)
    cc = jnp.sum(ac * ac, axis=0, keepdims=True)
    gce = cc - e
    gae = aa - e
    ftab_ref[9:10, :] = jnp.sum(ab * a, axis=0, keepdims=True)
    ftab_ref[10:11, :] = jnp.sum(ac * a, axis=0, keepdims=True)
    ftab_ref[11:12, :] = aa
    ftab_ref[12:13, :] = e
    ftab_ref[13:14, :] = cc
    ftab_ref[14:15, :] = gce
    ftab_ref[15:16, :] = gae
    ftab_ref[16:17, :] = gae * cc + e * gce
    ftab_ref[17:24, :] = jnp.zeros((7, _F), _f32)
    gmat_ref[0:3, 0:_F] = ab
    gmat_ref[0:3, _F : 2 * _F] = ac
    gmat_ref[3:8, :] = jnp.zeros((5, 2 * _F), _f32)


def _main_kernel(qp_ref, ftab_ref, gmat_ref, tgtt_ref, tgq_ref, out_ref):
    i = pl.program_id(0)

    p = qp_ref[:, :]
    px = p[:, 0:1]
    py = p[:, 1:2]
    pz = p[:, 2:3]

    gres = lax.dot_general(
        p,
        gmat_ref[0:3, :],
        (((1,), (0,)), ((), ())),
        preferred_element_type=_f32,
    )
    d1 = gres[:, 0:_F] - ftab_ref[9:10, :]
    d2_ = gres[:, _F : 2 * _F] - ftab_ref[10:11, :]
    aa = ftab_ref[11:12, :]
    e = ftab_ref[12:13, :]
    cc = ftab_ref[13:14, :]
    gce = ftab_ref[14:15, :]
    gae = ftab_ref[15:16, :]
    det = ftab_ref[16:17, :]
    d3 = d1 - aa
    d4 = d2_ - e
    d5 = d1 - e
    d6 = d2_ - cc

    d21 = d2_ - d1
    vb = gce * d1 - e * d21
    vc = gae * d2_ + e * d21
    va = det - (gce * d1 + gae * d2_)
    eps = 1e-12
    rcp = 1.0 / ((va + vb) + vc + eps)
    v = vb * rcp
    w = vc * rcp
    s1 = d4 - d3
    s2 = d5 - d6
    tbc = s1 / ((s1 + s2) + eps)
    m = (va <= 0) & (s1 >= 0) & (s2 >= 0)
    v = jnp.where(m, 1.0 - tbc, v)
    w = jnp.where(m, tbc, w)
    tac = d2_ / ((d2_ - d6) + eps)
    m = (vb <= 0) & (d2_ >= 0) & (d6 <= 0)
    v = jnp.where(m, 0.0, v)
    w = jnp.where(m, tac, w)
    m = (d6 >= 0) & (d5 <= d6)
    v = jnp.where(m, 0.0, v)
    w = jnp.where(m, 1.0, w)
    tab = d1 / ((d1 - d3) + eps)
    m = (vc <= 0) & (d1 >= 0) & (d3 <= 0)
    v = jnp.where(m, tab, v)
    w = jnp.where(m, 0.0, w)
    m = (d3 >= 0) & (d4 <= d3)
    v = jnp.where(m, 1.0, v)
    w = jnp.where(m, 0.0, w)
    m = (d1 <= 0) & (d2_ <= 0)
    v = jnp.where(m, 0.0, v)
    w = jnp.where(m, 0.0, w)

    clx = ftab_ref[0:1, :] + (v * ftab_ref[3:4, :] + w * ftab_ref[6:7, :])
    cly = ftab_ref[1:2, :] + (v * ftab_ref[4:5, :] + w * ftab_ref[7:8, :])
    clz = ftab_ref[2:3, :] + (v * ftab_ref[5:6, :] + w * ftab_ref[8:9, :])
    ddx = px - clx
    ddy = py - cly
    ddz = pz - clz
    dsq = (ddx * ddx + ddy * ddy) + ddz * ddz

    ids_f = lax.broadcasted_iota(_i32, (_BQ, _F), 1)
    packed_a = jnp.bitwise_or(
        jnp.bitwise_and(lax.bitcast_convert_type(dsq, _i32), _i32(-2048)), ids_f
    )
    ma = jnp.min(packed_a, axis=1, keepdims=True)
    eqa = ids_f == jnp.bitwise_and(ma, _i32(2047))
    bcx = jnp.sum(jnp.where(eqa, clx, 0.0), axis=1, keepdims=True)
    bcy = jnp.sum(jnp.where(eqa, cly, 0.0), axis=1, keepdims=True)
    bcz = jnp.sum(jnp.where(eqa, clz, 0.0), axis=1, keepdims=True)

    dsx = px - bcx
    dsy = py - bcy
    dsz = pz - bcz
    t0 = dsx + 1e-10
    t1 = dsy + 1e-10
    t2 = dsz + 1e-10
    udf_s = jnp.sqrt((t0 * t0 + t1 * t1) + t2 * t2)

    gt = lax.dot_general(
        p,
        tgtt_ref[0:3, :],
        (((1,), (0,)), ((), ())),
        preferred_element_type=_f32,
    )
    pn = (px * px + py * py) + pz * pz
    tx = tgtt_ref[0:1, :]
    ty = tgtt_ref[1:2, :]
    tz = tgtt_ref[2:3, :]
    tn = (tx * tx + ty * ty) + tz * tz
    d2t = (pn - 2.0 * gt) + tn

    ids_t = lax.broadcasted_iota(_i32, (_BQ, _T), 1)
    packed = jnp.bitwise_or(
        jnp.bitwise_and(lax.bitcast_convert_type(d2t, _i32), _i32(-4096)), ids_t
    )
    gm = jnp.minimum(
        jnp.minimum(packed[:, 0:_G], packed[:, _G : 2 * _G]),
        jnp.minimum(packed[:, 2 * _G : 3 * _G], packed[:, 3 * _G : 4 * _G]),
    )

    odiv = lax.broadcasted_iota(_i32, (_BQ, 3 * _NO), 1) // 3
    ids_g = lax.broadcasted_iota(_i32, (_BQ, _G), 1)
    acc = jnp.zeros((_BQ, 3 * _NO), _f32)
    s = jnp.zeros((_BQ, 1), _f32)
    for k in range(_K):
        mk = jnp.min(gm, axis=1, keepdims=True)
        eq = ids_g == jnp.bitwise_and(mk, _i32(_G - 1))
        g1 = eq.astype(_f32)
        tmpk = lax.dot_general(
            g1,
            tgq_ref[:, :],
            (((1,), (0,)), ((), ())),
            preferred_element_type=_f32,
        )
        dk = lax.bitcast_convert_type(jnp.bitwise_and(mk, _i32(-4096)), _f32)
        wk = 1.0 / (dk + 1e-8)
        s = s + wk
        ok = jnp.right_shift(jnp.bitwise_and(mk, _i32(4095)), _i32(10))
        acc = acc + jnp.where(odiv == ok, tmpk * wk, 0.0)
        if k < _K - 1:
            gm = jnp.where(eq, _IMAX, gm)

    spx = ((acc[:, 0:1] + acc[:, 3:4]) + (acc[:, 6:7] + acc[:, 9:10])) / s
    spy = ((acc[:, 1:2] + acc[:, 4:5]) + (acc[:, 7:8] + acc[:, 10:11])) / s
    spz = ((acc[:, 2:3] + acc[:, 5:6]) + (acc[:, 8:9] + acc[:, 11:12])) / s
    dtx = px - spx
    dty = py - spy
    dtz = pz - spz
    r0 = dtx + 1e-10
    r1 = dty + 1e-10
    r2 = dtz + 1e-10
    udf_t = jnp.sqrt((r0 * r0 + r1 * r1) + r2 * r2)

    err = (
        (jnp.abs(dsx - dtx) + jnp.abs(dsy - dty)) + jnp.abs(dsz - dtz)
    ) + jnp.abs(udf_s - udf_t)
    tot = jnp.sum(err)

    ri = lax.broadcasted_iota(_i32, (8, 128), 0)
    ci = lax.broadcasted_iota(_i32, (8, 128), 1)
    upd = jnp.where((ri == 0) & (ci == 0), tot, 0.0)

    @pl.when(i == 0)
    def _():
        out_ref[:, :] = jnp.zeros((8, 128), _f32)

    out_ref[:, :] += upd


@functools.partial(jax.jit, static_argnames=("interpret",))
def kernel(src_v, src_f, tgt_points, noise, interpret=False):
    src_ft = jnp.zeros((8, _F), _i32).at[0:3, :].set(src_f.T)

    ftab, gmat = pl.pallas_call(
        _prep_kernel,
        out_shape=[
            jax.ShapeDtypeStruct((24, _F), _f32),
            jax.ShapeDtypeStruct((8, 2 * _F), _f32),
        ],
        interpret=interpret,
    )(src_v, src_ft)

    center = ftab[0:3, :] + (ftab[3:6, :] + ftab[6:9, :]) / 3.0
    qp = jnp.concatenate(
        [(tgt_points[:, None, :] + _STD * noise).reshape(-1, 3), center.T], axis=0
    )
    tgtt = jnp.zeros((8, _T), _f32).at[0:3, :].set(tgt_points.T)
    tgq = jnp.transpose(tgt_points.reshape(_NO, _G, 3), (1, 0, 2)).reshape(
        _G, 3 * _NO
    )

    acc = pl.pallas_call(
        _main_kernel,
        grid=(_Q // _BQ,),
        in_specs=[
            pl.BlockSpec((_BQ, 3), lambda i: (i, 0)),
            pl.BlockSpec((24, _F), lambda i: (0, 0)),
            pl.BlockSpec((8, 2 * _F), lambda i: (0, 0)),
            pl.BlockSpec((8, _T), lambda i: (0, 0)),
            pl.BlockSpec((_G, 3 * _NO), lambda i: (0, 0)),
        ],
        out_specs=pl.BlockSpec((8, 128), lambda i: (0, 0)),
        out_shape=jax.ShapeDtypeStruct((8, 128), _f32),
        interpret=interpret,
    )(qp, ftab, gmat, tgtt, tgq)

    return acc[0, 0] / _Q
